# baseline (speedup 1.0000x reference)
import functools
import numpy as np
import jax
import jax.numpy as jnp
from jax import lax
from jax.experimental import pallas as pl
from jax.experimental.pallas import tpu as pltpu
from jax.experimental.pallas import tpu_sc as plsc

_B = 4096
_F = 26
_V = 100000
_D = 64
_NF = _F + 1
_NUM_INTER = (_NF * (_NF - 1)) // 2
_B_BLK = 256
_NW = 32
_CH = 128


def _sc_gather(table, flat_idx):
  R = flat_idx.shape[0]
  D = table.shape[1]
  r_per_w = R // _NW
  n_ch = r_per_w // _CH
  mesh = plsc.VectorSubcoreMesh(core_axis_name="c", subcore_axis_name="s")

  @functools.partial(
      pl.kernel,
      mesh=mesh,
      out_type=jax.ShapeDtypeStruct((R, D), jnp.float32),
      scratch_types=[
          pltpu.VMEM((_CH,), jnp.int32),
          pltpu.VMEM((_CH, D), jnp.float32),
          pltpu.SemaphoreType.DMA,
      ],
      compiler_params=pltpu.CompilerParams(use_tc_tiling_on_sc=False),
  )
  def gather_k(table_hbm, idx_hbm, out_hbm, idx_v, rows_v, sem):
    wid = lax.axis_index("s") * 2 + lax.axis_index("c")
    base = wid * r_per_w

    def body(c, carry):
      b0 = base + c * _CH
      pltpu.sync_copy(idx_hbm.at[pl.ds(b0, _CH)], idx_v)
      pltpu.async_copy(table_hbm.at[idx_v], rows_v, sem).wait()
      pltpu.sync_copy(rows_v, out_hbm.at[pl.ds(b0, _CH)])
      return carry

    lax.fori_loop(0, n_ch, body, 0)

  return gather_k(table, flat_idx)


def _tc_body(emb_ref, xdt_ref, bw0, bb0, bw1, bb1, bw2, bb2,
             tw0p, tb0, tw1, tb1, tw2, tb2, out_ref, tt_ref):
  c00 = (((0,), (0,)), ((), ()))
  f32 = jnp.float32
  h = jnp.maximum(
      lax.dot_general(bw0[...], xdt_ref[...], c00, preferred_element_type=f32)
      + bb0[...], 0.0)
  h = jnp.maximum(
      lax.dot_general(bw1[...], h, c00, preferred_element_type=f32)
      + bb1[...], 0.0)
  dot_t = (lax.dot_general(bw2[...], h, c00, preferred_element_type=f32)
           + bb2[...])

  et = jnp.transpose(emb_ref[...])

  tt_ref[_NUM_INTER + 1:_NUM_INTER + 1 + _D, :] = dot_t
  tt_ref[_NUM_INTER:_NUM_INTER + 1, :] = jnp.zeros((1, _B_BLK), f32)

  slices = [et[i * _D:(i + 1) * _D, :] for i in range(_F)] + [dot_t]
  p = 0
  for i in range(_NF):
    ai = slices[i]
    for j in range(i + 1, _NF):
      s = jnp.sum(ai * slices[j], axis=0, keepdims=True)
      tt_ref[p:p + 1, :] = s
      p += 1

  tt = tt_ref[...]
  h2 = jnp.maximum(
      lax.dot_general(tt, tw0p[...], c00, preferred_element_type=f32)
      + tb0[...], 0.0)
  h3 = jnp.maximum(
      jnp.dot(h2, tw1[...], preferred_element_type=f32) + tb1[...], 0.0)
  out_ref[...] = jnp.dot(h3, tw2[...], preferred_element_type=f32) + tb2[...]


def _tc_dense(embeds, xdt, bw0, bb0, bw1, bb1, bw2, bb2,
              tw0p, tb0, tw1, tb1, tw2, tb2):
  n_blk = _B // _B_BLK
  full = lambda shape: pl.BlockSpec(shape, lambda i: (0,) * len(shape))
  return pl.pallas_call(
      _tc_body,
      grid=(n_blk,),
      in_specs=[
          pl.BlockSpec((_B_BLK, _F * _D), lambda i: (i, 0)),
          pl.BlockSpec((13, _B_BLK), lambda i: (0, i)),
          full(bw0.shape), full(bb0.shape), full(bw1.shape), full(bb1.shape),
          full(bw2.shape), full(bb2.shape), full(tw0p.shape), full(tb0.shape),
          full(tw1.shape), full(tb1.shape), full(tw2.shape), full(tb2.shape),
      ],
      out_specs=pl.BlockSpec((_B_BLK, 1), lambda i: (i, 0)),
      out_shape=jax.ShapeDtypeStruct((_B, 1), jnp.float32),
      scratch_shapes=[pltpu.VMEM((_NUM_INTER + 1 + _D, _B_BLK), jnp.float32)],
  )(embeds, xdt, bw0, bb0, bw1, bb1, bw2, bb2,
    tw0p, tb0, tw1, tb1, tw2, tb2)


def kernel(x_sparse, x_dense, emb, bw0, bb0, bw1, bb1, bw2, bb2,
           tw0, tb0, tw1, tb1, tw2, tb2):
  flat_idx = (x_sparse + jnp.arange(_F, dtype=jnp.int32)[None, :] * _V).reshape(-1)
  table = emb.reshape(_F * _V, _D)
  embeds = _sc_gather(table, flat_idx).reshape(_B, _F * _D)

  tw0p = jnp.concatenate(
      [tw0[:_NUM_INTER], jnp.zeros((1, tw0.shape[1]), tw0.dtype),
       tw0[_NUM_INTER:]], axis=0)
  out = _tc_dense(
      embeds, x_dense.T,
      bw0, bb0.reshape(-1, 1), bw1, bb1.reshape(-1, 1), bw2, bb2.reshape(-1, 1),
      tw0p, tb0.reshape(1, -1), tw1, tb1.reshape(1, -1), tw2, tb2.reshape(1, -1))
  return out
